# baseline (speedup 1.0000x reference)
import functools

import jax
import jax.numpy as jnp
from jax import lax
from jax.experimental import pallas as pl
from jax.experimental.pallas import tpu as pltpu
from jax.experimental.pallas import tpu_sc as plsc

VOCAB = 100000
MAXLEN = 200
EMBED = 300
BATCH = 4096

NC, NS, L = 2, 16, 16
NW = NC * NS

CHUNK = 40
TOK = BATCH * MAXLEN
PER_W = TOK // NW
N_CHUNK = PER_W // CHUNK
JMOD = MAXLEN // CHUNK
NFULL = EMBED // L
TAIL = EMBED - L
PADW = (NFULL + 1) * L
PK = CHUNK * EMBED

_mesh = plsc.VectorSubcoreMesh(core_axis_name="c", subcore_axis_name="s")


@functools.partial(
    pl.kernel,
    mesh=_mesh,
    compiler_params=pltpu.CompilerParams(use_tc_tiling_on_sc=False),
    out_type=jax.ShapeDtypeStruct((TOK * EMBED,), jnp.float32),
    scratch_types=[
        pltpu.VMEM((MAXLEN, PADW), jnp.float32),
        pltpu.VMEM((4, CHUNK), jnp.int32),
        pltpu.VMEM((2, CHUNK, PADW), jnp.float32),
        pltpu.VMEM((2, PK), jnp.float32),
        pltpu.SemaphoreType.DMA,
        pltpu.SemaphoreType.DMA,
        pltpu.SemaphoreType.DMA,
        pltpu.SemaphoreType.DMA,
        pltpu.SemaphoreType.DMA,
        pltpu.SemaphoreType.DMA,
        pltpu.SemaphoreType.DMA,
        pltpu.SemaphoreType.DMA,
    ],
)
def _emb(tok_hbm, idx_hbm, posadj_hbm, out_hbm, pos_v, idx_v, rows_v, pk_v,
         g0, g1, o0, o1, i0, i1, i2, i3):
    gsem = (g0, g1)
    osem = (o0, o1)
    isem = (i0, i1, i2, i3)
    wid = lax.axis_index("s") * NC + lax.axis_index("c")
    base = wid * PER_W
    pltpu.sync_copy(posadj_hbm, pos_v)

    def start_idx(i, jb):
        pltpu.async_copy(idx_hbm.at[pl.ds(base + i * CHUNK, CHUNK)],
                         idx_v.at[jb], isem[jb])

    def wait_idx(jb):
        pltpu.make_async_copy(idx_hbm.at[pl.ds(0, CHUNK)], idx_v.at[jb],
                              isem[jb]).wait()

    def start_gather(jb, b):
        pltpu.async_copy(tok_hbm.at[idx_v.at[jb]], rows_v.at[b], gsem[b])

    def wait_gather(b):
        pltpu.make_async_copy(tok_hbm.at[pl.ds(0, CHUNK)], rows_v.at[b],
                              gsem[b]).wait()

    def start_out(i, b):
        pltpu.async_copy(pk_v.at[b], out_hbm.at[pl.ds((base + i * CHUNK) * EMBED, PK)],
                         osem[b])

    def wait_out(b):
        pltpu.make_async_copy(pk_v.at[b], out_hbm.at[pl.ds(0, PK)],
                              osem[b]).wait()

    def pack(i, b):
        p0 = lax.rem(i, JMOD) * CHUNK

        def row_body(r, rc):
            p = p0 + r
            o = r * EMBED
            for k in range(NFULL):
                pk_v[b, pl.ds(o + k * L, L)] = (
                    rows_v[b, r, pl.ds(k * L, L)] + pos_v[p, pl.ds(k * L, L)])
            pk_v[b, pl.ds(o + TAIL, L)] = (
                rows_v[b, r, pl.ds(TAIL, L)] + pos_v[p, pl.ds(NFULL * L, L)])
            return rc

        lax.fori_loop(0, CHUNK, row_body, 0)

    def step(i, b, ib, gather_next, prefetch_idx, wait_o):
        if gather_next:
            nib = (ib + 1) % 4
            wait_idx(nib)
            start_gather(nib, 1 - b)
        wait_gather(b)
        if prefetch_idx:
            start_idx(i + 2, (ib + 2) % 4)
        if wait_o:
            wait_out(b)
        pack(i, b)
        start_out(i, b)

    start_idx(0, 0)
    start_idx(1, 1)
    wait_idx(0)
    start_gather(0, 0)
    step(0, 0, 0, True, True, False)
    step(1, 1, 1, True, True, False)

    def quad(q, carry):
        i0_ = 2 + 4 * q
        step(i0_, 0, 2, True, True, True)
        step(i0_ + 1, 1, 3, True, True, True)
        step(i0_ + 2, 0, 0, True, True, True)
        step(i0_ + 3, 1, 1, True, True, True)
        return carry

    lax.fori_loop(0, (N_CHUNK - 4) // 4, quad, 0)

    step(N_CHUNK - 2, 0, 2, True, False, True)
    step(N_CHUNK - 1, 1, 3, False, False, True)
    wait_out(0)
    wait_out(1)


def kernel(x, token_table, pos_table):
    idx = x.reshape(-1).astype(jnp.int32)
    tok_pad = jnp.pad(token_table, ((0, 0), (0, PADW - EMBED)))
    pos_adj = jnp.concatenate([pos_table[:, : NFULL * L], pos_table[:, TAIL:]], axis=1)
    out = _emb(tok_pad, idx, pos_adj)
    return out.reshape(BATCH, MAXLEN, EMBED)


# speedup vs baseline: 1.3716x; 1.3716x over previous
import functools

import jax
import jax.numpy as jnp
from jax import lax
from jax.experimental import pallas as pl
from jax.experimental.pallas import tpu as pltpu
from jax.experimental.pallas import tpu_sc as plsc

VOCAB = 100000
MAXLEN = 200
EMBED = 300
BATCH = 4096

NC, NS, L = 2, 16, 16
NW = NC * NS

CHUNK = 40
TOK = BATCH * MAXLEN
PER_W = TOK // NW
N_CHUNK = PER_W // CHUNK
ROWS_W = BATCH // NW
JMOD = MAXLEN // CHUNK
NFULL = EMBED // L
TAIL = EMBED - L
PADW = 384
POSW = (NFULL + 1) * L

_mesh = plsc.VectorSubcoreMesh(core_axis_name="c", subcore_axis_name="s")


@functools.partial(
    pl.kernel,
    mesh=_mesh,
    compiler_params=pltpu.CompilerParams(use_tc_tiling_on_sc=True),
    out_type=jax.ShapeDtypeStruct((BATCH, MAXLEN, EMBED), jnp.float32),
    scratch_types=[
        pltpu.VMEM((MAXLEN * POSW,), jnp.float32),
        pltpu.VMEM((4, CHUNK), jnp.int32),
        pltpu.VMEM((2, CHUNK, PADW), jnp.float32),
        pltpu.VMEM((2, CHUNK, EMBED), jnp.float32),
        pltpu.SemaphoreType.DMA,
        pltpu.SemaphoreType.DMA,
        pltpu.SemaphoreType.DMA,
        pltpu.SemaphoreType.DMA,
        pltpu.SemaphoreType.DMA,
        pltpu.SemaphoreType.DMA,
        pltpu.SemaphoreType.DMA,
        pltpu.SemaphoreType.DMA,
    ],
)
def _emb(tok_hbm, idx_hbm, posadj_hbm, out_hbm, pos_v, idx_v, rows_v, pk_v,
         g0, g1, o0, o1, i0, i1, i2, i3):
    gsem = (g0, g1)
    osem = (o0, o1)
    isem = (i0, i1, i2, i3)
    wid = lax.axis_index("s") * NC + lax.axis_index("c")
    base = wid * PER_W
    row0 = wid * ROWS_W
    pltpu.sync_copy(posadj_hbm, pos_v)

    def start_idx(i, jb):
        pltpu.async_copy(idx_hbm.at[pl.ds(base + i * CHUNK, CHUNK)],
                         idx_v.at[jb], isem[jb])

    def wait_idx(jb):
        pltpu.make_async_copy(idx_hbm.at[pl.ds(0, CHUNK)], idx_v.at[jb],
                              isem[jb]).wait()

    def start_gather(jb, b):
        pltpu.async_copy(tok_hbm.at[idx_v.at[jb]], rows_v.at[b], gsem[b])

    def wait_gather(b):
        pltpu.make_async_copy(tok_hbm.at[pl.ds(0, CHUNK)], rows_v.at[b],
                              gsem[b]).wait()

    def start_out(i, b):
        bb = row0 + lax.div(i, JMOD)
        t0 = lax.rem(i, JMOD) * CHUNK
        pltpu.async_copy(pk_v.at[b], out_hbm.at[bb, pl.ds(t0, CHUNK)], osem[b])

    def wait_out(b):
        pltpu.make_async_copy(pk_v.at[b], out_hbm.at[0, pl.ds(0, CHUNK)],
                              osem[b]).wait()

    def pack(i, b):
        p0 = lax.rem(i, JMOD) * CHUNK

        def row_body(r, rc):
            po = (p0 + r) * POSW
            pk_v[b, r, pl.ds(TAIL, L)] = (
                rows_v[b, r, pl.ds(TAIL, L)] + pos_v[pl.ds(po + NFULL * L, L)])
            for k in range(NFULL):
                pk_v[b, r, pl.ds(k * L, L)] = (
                    rows_v[b, r, pl.ds(k * L, L)] + pos_v[pl.ds(po + k * L, L)])
            return rc

        lax.fori_loop(0, CHUNK, row_body, 0)

    def step(i, b, ib, gather_next, prefetch_idx, wait_o):
        if gather_next:
            nib = (ib + 1) % 4
            wait_idx(nib)
            start_gather(nib, 1 - b)
        wait_gather(b)
        if prefetch_idx:
            start_idx(i + 2, (ib + 2) % 4)
        if wait_o:
            wait_out(b)
        pack(i, b)
        start_out(i, b)

    start_idx(0, 0)
    start_idx(1, 1)
    wait_idx(0)
    start_gather(0, 0)
    step(0, 0, 0, True, True, False)
    step(1, 1, 1, True, True, False)

    def quad(q, carry):
        i0_ = 2 + 4 * q
        step(i0_, 0, 2, True, True, True)
        step(i0_ + 1, 1, 3, True, True, True)
        step(i0_ + 2, 0, 0, True, True, True)
        step(i0_ + 3, 1, 1, True, True, True)
        return carry

    lax.fori_loop(0, (N_CHUNK - 4) // 4, quad, 0)

    step(N_CHUNK - 2, 0, 2, True, False, True)
    step(N_CHUNK - 1, 1, 3, False, False, True)
    wait_out(0)
    wait_out(1)


def kernel(x, token_table, pos_table):
    idx = x.reshape(-1).astype(jnp.int32)
    tok_pad = jnp.pad(token_table, ((0, 0), (0, PADW - EMBED)))
    pos_adj = jnp.concatenate([pos_table[:, : NFULL * L], pos_table[:, TAIL:]],
                              axis=1).reshape(-1)
    return _emb(tok_pad, idx, pos_adj)


# speedup vs baseline: 1.4662x; 1.0690x over previous
import functools

import jax
import jax.numpy as jnp
from jax import lax
from jax.experimental import pallas as pl
from jax.experimental.pallas import tpu as pltpu
from jax.experimental.pallas import tpu_sc as plsc

VOCAB = 100000
MAXLEN = 200
EMBED = 300
BATCH = 4096

NC, NS, L = 2, 16, 16
NW = NC * NS

CHUNK = 40
TOK = BATCH * MAXLEN
PER_W = TOK // NW
N_CHUNK = PER_W // CHUNK
ROWS_W = BATCH // NW
JMOD = MAXLEN // CHUNK
NFULL = EMBED // L
TAIL = EMBED - L
PADW = 384
POSW = (NFULL + 1) * L

_mesh = plsc.VectorSubcoreMesh(core_axis_name="c", subcore_axis_name="s")


@functools.partial(
    pl.kernel,
    mesh=_mesh,
    compiler_params=pltpu.CompilerParams(use_tc_tiling_on_sc=True),
    out_type=jax.ShapeDtypeStruct((BATCH, MAXLEN, PADW), jnp.float32),
    scratch_types=[
        pltpu.VMEM((MAXLEN * POSW,), jnp.float32),
        pltpu.VMEM((4, CHUNK), jnp.int32),
        pltpu.VMEM((4, CHUNK, PADW), jnp.float32),
        pltpu.SemaphoreType.DMA, pltpu.SemaphoreType.DMA,
        pltpu.SemaphoreType.DMA, pltpu.SemaphoreType.DMA,
        pltpu.SemaphoreType.DMA, pltpu.SemaphoreType.DMA,
        pltpu.SemaphoreType.DMA, pltpu.SemaphoreType.DMA,
        pltpu.SemaphoreType.DMA, pltpu.SemaphoreType.DMA,
        pltpu.SemaphoreType.DMA, pltpu.SemaphoreType.DMA,
    ],
)
def _emb(tok_hbm, idx_hbm, posadj_hbm, out_hbm, pos_v, idx_v, rows_v,
         g0, g1, g2, g3, o0, o1, o2, o3, i0, i1, i2, i3):
    gsem = (g0, g1, g2, g3)
    osem = (o0, o1, o2, o3)
    isem = (i0, i1, i2, i3)
    wid = lax.axis_index("s") * NC + lax.axis_index("c")
    base = wid * PER_W
    row0 = wid * ROWS_W
    pltpu.sync_copy(posadj_hbm, pos_v)

    def start_idx(i, jb):
        pltpu.async_copy(idx_hbm.at[pl.ds(base + i * CHUNK, CHUNK)],
                         idx_v.at[jb], isem[jb])

    def wait_idx(jb):
        pltpu.make_async_copy(idx_hbm.at[pl.ds(0, CHUNK)], idx_v.at[jb],
                              isem[jb]).wait()

    def start_gather(jb, b):
        pltpu.async_copy(tok_hbm.at[idx_v.at[jb]], rows_v.at[b], gsem[b])

    def wait_gather(b):
        pltpu.make_async_copy(tok_hbm.at[pl.ds(0, CHUNK)], rows_v.at[b],
                              gsem[b]).wait()

    def start_out(i, b):
        bb = row0 + lax.div(i, JMOD)
        t0 = lax.rem(i, JMOD) * CHUNK
        pltpu.async_copy(rows_v.at[b], out_hbm.at[bb, pl.ds(t0, CHUNK)], osem[b])

    def wait_out(b):
        pltpu.make_async_copy(rows_v.at[b], out_hbm.at[0, pl.ds(0, CHUNK)],
                              osem[b]).wait()

    def add_pos(i, b):
        p0 = lax.rem(i, JMOD) * CHUNK

        def row_body(r, rc):
            po = (p0 + r) * POSW
            for k in range(NFULL + 1):
                rows_v[b, r, pl.ds(k * L, L)] += pos_v[pl.ds(po + k * L, L)]
            return rc

        lax.fori_loop(0, CHUNK, row_body, 0)

    def step(i, b, gather2, prefetch3, wait_o):
        if gather2:
            nb = (b + 2) % 4
            wait_idx(nb)
            if wait_o:
                wait_out(nb)
            start_gather(nb, nb)
        if prefetch3:
            start_idx(i + 3, (b + 3) % 4)
        wait_gather(b)
        add_pos(i, b)
        start_out(i, b)

    start_idx(0, 0)
    start_idx(1, 1)
    start_idx(2, 2)
    wait_idx(0)
    start_gather(0, 0)
    wait_idx(1)
    start_gather(1, 1)
    step(0, 0, True, True, False)
    step(1, 1, True, True, False)

    def quad(q, carry):
        i0_ = 2 + 4 * q
        step(i0_, 2, True, True, True)
        step(i0_ + 1, 3, True, True, True)
        step(i0_ + 2, 0, True, True, True)
        step(i0_ + 3, 1, True, True, True)
        return carry

    lax.fori_loop(0, (N_CHUNK - 8) // 4, quad, 0)

    step(N_CHUNK - 6, 2, True, True, True)
    step(N_CHUNK - 5, 3, True, True, True)
    step(N_CHUNK - 4, 0, True, True, True)
    step(N_CHUNK - 3, 1, True, False, True)
    step(N_CHUNK - 2, 2, False, False, False)
    step(N_CHUNK - 1, 3, False, False, False)
    wait_out(0)
    wait_out(1)
    wait_out(2)
    wait_out(3)


def kernel(x, token_table, pos_table):
    idx = x.reshape(-1).astype(jnp.int32)
    tok_pad = jnp.pad(token_table, ((0, 0), (0, PADW - EMBED)))
    pos_adj = jnp.pad(pos_table, ((0, 0), (0, POSW - EMBED))).reshape(-1)
    out = _emb(tok_pad, idx, pos_adj)
    return out[:, :, :EMBED]


# speedup vs baseline: 1.5240x; 1.0394x over previous
import functools

import jax
import jax.numpy as jnp
from jax import lax
from jax.experimental import pallas as pl
from jax.experimental.pallas import tpu as pltpu
from jax.experimental.pallas import tpu_sc as plsc

VOCAB = 100000
MAXLEN = 200
EMBED = 300
BATCH = 4096

NC, NS, L = 2, 16, 16
NW = NC * NS

CHUNK = 40
TOK = BATCH * MAXLEN
PER_W = TOK // NW
N_CHUNK = PER_W // CHUNK
ROWS_W = BATCH // NW
JMOD = MAXLEN // CHUNK
NFULL = EMBED // L
TAIL = EMBED - L
PADW = 384
POSW = (NFULL + 1) * L

_mesh = plsc.VectorSubcoreMesh(core_axis_name="c", subcore_axis_name="s")


@functools.partial(
    pl.kernel,
    mesh=_mesh,
    compiler_params=pltpu.CompilerParams(use_tc_tiling_on_sc=True),
    out_type=jax.ShapeDtypeStruct((BATCH, MAXLEN, PADW), jnp.float32),
    scratch_types=[
        pltpu.VMEM((MAXLEN * POSW,), jnp.float32),
        pltpu.VMEM((4, CHUNK), jnp.int32),
        pltpu.VMEM((4, CHUNK, PADW), jnp.float32),
        pltpu.SemaphoreType.DMA, pltpu.SemaphoreType.DMA,
        pltpu.SemaphoreType.DMA, pltpu.SemaphoreType.DMA,
        pltpu.SemaphoreType.DMA, pltpu.SemaphoreType.DMA,
        pltpu.SemaphoreType.DMA, pltpu.SemaphoreType.DMA,
        pltpu.SemaphoreType.DMA, pltpu.SemaphoreType.DMA,
        pltpu.SemaphoreType.DMA, pltpu.SemaphoreType.DMA,
    ],
)
def _emb(tok_hbm, idx_hbm, posadj_hbm, out_hbm, pos_v, idx_v, rows_v,
         g0, g1, g2, g3, o0, o1, o2, o3, i0, i1, i2, i3):
    gsem = (g0, g1, g2, g3)
    osem = (o0, o1, o2, o3)
    isem = (i0, i1, i2, i3)
    wid = lax.axis_index("s") * NC + lax.axis_index("c")
    base = wid * PER_W
    row0 = wid * ROWS_W
    pltpu.sync_copy(posadj_hbm, pos_v)

    def start_idx(i, jb):
        pltpu.async_copy(idx_hbm.at[pl.ds(base + i * CHUNK, CHUNK)],
                         idx_v.at[jb], isem[jb])

    def wait_idx(jb):
        pltpu.make_async_copy(idx_hbm.at[pl.ds(0, CHUNK)], idx_v.at[jb],
                              isem[jb]).wait()

    def start_gather(jb, b):
        pltpu.async_copy(tok_hbm.at[idx_v.at[jb]], rows_v.at[b], gsem[b])

    def wait_gather(b):
        pltpu.make_async_copy(tok_hbm.at[pl.ds(0, CHUNK)], rows_v.at[b],
                              gsem[b]).wait()

    def start_out(i, b):
        bb = row0 + lax.div(i, JMOD)
        t0 = lax.rem(i, JMOD) * CHUNK
        pltpu.async_copy(rows_v.at[b], out_hbm.at[bb, pl.ds(t0, CHUNK)], osem[b])

    def wait_out(b):
        pltpu.make_async_copy(rows_v.at[b], out_hbm.at[0, pl.ds(0, CHUNK)],
                              osem[b]).wait()

    def add_pos(i, b):
        p0 = lax.rem(i, JMOD) * CHUNK

        def row_body(h, rc):
            r = 2 * h
            po = (p0 + r) * POSW
            for k in range(NFULL + 1):
                rows_v[b, r, pl.ds(k * L, L)] += pos_v[pl.ds(po + k * L, L)]
            for k in range(NFULL + 1):
                rows_v[b, r + 1, pl.ds(k * L, L)] += pos_v[pl.ds(po + POSW + k * L, L)]
            return rc

        lax.fori_loop(0, CHUNK // 2, row_body, 0)

    def step(i, b, gather2, prefetch3, wait_o):
        if gather2:
            nb = (b + 2) % 4
            wait_idx(nb)
            if wait_o:
                wait_out(nb)
            start_gather(nb, nb)
        if prefetch3:
            start_idx(i + 3, (b + 3) % 4)
        wait_gather(b)
        add_pos(i, b)
        start_out(i, b)

    start_idx(0, 0)
    start_idx(1, 1)
    start_idx(2, 2)
    wait_idx(0)
    start_gather(0, 0)
    wait_idx(1)
    start_gather(1, 1)
    step(0, 0, True, True, False)
    step(1, 1, True, True, False)

    def quad(q, carry):
        i0_ = 2 + 4 * q
        step(i0_, 2, True, True, True)
        step(i0_ + 1, 3, True, True, True)
        step(i0_ + 2, 0, True, True, True)
        step(i0_ + 3, 1, True, True, True)
        return carry

    lax.fori_loop(0, (N_CHUNK - 8) // 4, quad, 0)

    step(N_CHUNK - 6, 2, True, True, True)
    step(N_CHUNK - 5, 3, True, True, True)
    step(N_CHUNK - 4, 0, True, True, True)
    step(N_CHUNK - 3, 1, True, False, True)
    step(N_CHUNK - 2, 2, False, False, False)
    step(N_CHUNK - 1, 3, False, False, False)
    wait_out(0)
    wait_out(1)
    wait_out(2)
    wait_out(3)


def kernel(x, token_table, pos_table):
    idx = x.reshape(-1).astype(jnp.int32)
    tok_pad = jnp.pad(token_table, ((0, 0), (0, PADW - EMBED)))
    pos_adj = jnp.pad(pos_table, ((0, 0), (0, POSW - EMBED))).reshape(-1)
    out = _emb(tok_pad, idx, pos_adj)
    return out[:, :, :EMBED]


# speedup vs baseline: 1.5279x; 1.0026x over previous
import functools

import jax
import jax.numpy as jnp
from jax import lax
from jax.experimental import pallas as pl
from jax.experimental.pallas import tpu as pltpu
from jax.experimental.pallas import tpu_sc as plsc

VOCAB = 100000
MAXLEN = 200
EMBED = 300
BATCH = 4096

NC, NS, L = 2, 16, 16
NW = NC * NS

CHUNK = 40
TOK = BATCH * MAXLEN
PER_W = TOK // NW
N_CHUNK = PER_W // CHUNK
ROWS_W = BATCH // NW
JMOD = MAXLEN // CHUNK
NFULL = EMBED // L
TAIL = EMBED - L
PADW = 384
POSW = (NFULL + 1) * L

_mesh = plsc.VectorSubcoreMesh(core_axis_name="c", subcore_axis_name="s")


@functools.partial(
    pl.kernel,
    mesh=_mesh,
    compiler_params=pltpu.CompilerParams(use_tc_tiling_on_sc=True),
    out_type=jax.ShapeDtypeStruct((BATCH, MAXLEN, PADW), jnp.float32),
    scratch_types=[
        pltpu.VMEM((MAXLEN * POSW,), jnp.float32),
        pltpu.VMEM((4, CHUNK), jnp.int32),
        pltpu.VMEM((4, CHUNK, PADW), jnp.float32),
        pltpu.SemaphoreType.DMA, pltpu.SemaphoreType.DMA,
        pltpu.SemaphoreType.DMA, pltpu.SemaphoreType.DMA,
        pltpu.SemaphoreType.DMA, pltpu.SemaphoreType.DMA,
        pltpu.SemaphoreType.DMA, pltpu.SemaphoreType.DMA,
        pltpu.SemaphoreType.DMA, pltpu.SemaphoreType.DMA,
        pltpu.SemaphoreType.DMA, pltpu.SemaphoreType.DMA,
    ],
)
def _emb(tok_hbm, idx_hbm, posadj_hbm, out_hbm, pos_v, idx_v, rows_v,
         g0, g1, g2, g3, o0, o1, o2, o3, i0, i1, i2, i3):
    gsem = (g0, g1, g2, g3)
    osem = (o0, o1, o2, o3)
    isem = (i0, i1, i2, i3)
    wid = lax.axis_index("s") * NC + lax.axis_index("c")
    base = wid * PER_W
    row0 = wid * ROWS_W
    pltpu.sync_copy(posadj_hbm, pos_v)

    def start_idx(i, jb):
        pltpu.async_copy(idx_hbm.at[pl.ds(base + i * CHUNK, CHUNK)],
                         idx_v.at[jb], isem[jb])

    def wait_idx(jb):
        pltpu.make_async_copy(idx_hbm.at[pl.ds(0, CHUNK)], idx_v.at[jb],
                              isem[jb]).wait()

    def start_gather(jb, b):
        pltpu.async_copy(tok_hbm.at[idx_v.at[jb]], rows_v.at[b], gsem[b])

    def wait_gather(b):
        pltpu.make_async_copy(tok_hbm.at[pl.ds(0, CHUNK)], rows_v.at[b],
                              gsem[b]).wait()

    def start_out(i, b):
        bb = row0 + lax.div(i, JMOD)
        t0 = lax.rem(i, JMOD) * CHUNK
        pltpu.async_copy(rows_v.at[b], out_hbm.at[bb, pl.ds(t0, CHUNK)], osem[b])

    def wait_out(b):
        pltpu.make_async_copy(rows_v.at[b], out_hbm.at[0, pl.ds(0, CHUNK)],
                              osem[b]).wait()

    def add_pos(i, b):
        p0 = lax.rem(i, JMOD) * CHUNK

        def row_body(h, rc):
            r = 4 * h
            po = (p0 + r) * POSW
            for d in range(4):
                for k in range(NFULL + 1):
                    rows_v[b, r + d, pl.ds(k * L, L)] += (
                        pos_v[pl.ds(po + d * POSW + k * L, L)])
            return rc

        lax.fori_loop(0, CHUNK // 4, row_body, 0)

    def step(i, b, gather2, prefetch3, wait_o):
        if gather2:
            nb = (b + 2) % 4
            wait_idx(nb)
            if wait_o:
                wait_out(nb)
            start_gather(nb, nb)
        if prefetch3:
            start_idx(i + 3, (b + 3) % 4)
        wait_gather(b)
        add_pos(i, b)
        start_out(i, b)

    start_idx(0, 0)
    start_idx(1, 1)
    start_idx(2, 2)
    wait_idx(0)
    start_gather(0, 0)
    wait_idx(1)
    start_gather(1, 1)
    step(0, 0, True, True, False)
    step(1, 1, True, True, False)

    def quad(q, carry):
        i0_ = 2 + 4 * q
        step(i0_, 2, True, True, True)
        step(i0_ + 1, 3, True, True, True)
        step(i0_ + 2, 0, True, True, True)
        step(i0_ + 3, 1, True, True, True)
        return carry

    lax.fori_loop(0, (N_CHUNK - 8) // 4, quad, 0)

    step(N_CHUNK - 6, 2, True, True, True)
    step(N_CHUNK - 5, 3, True, True, True)
    step(N_CHUNK - 4, 0, True, True, True)
    step(N_CHUNK - 3, 1, True, False, True)
    step(N_CHUNK - 2, 2, False, False, False)
    step(N_CHUNK - 1, 3, False, False, False)
    wait_out(0)
    wait_out(1)
    wait_out(2)
    wait_out(3)


def kernel(x, token_table, pos_table):
    idx = x.reshape(-1).astype(jnp.int32)
    tok_pad = jnp.pad(token_table, ((0, 0), (0, PADW - EMBED)))
    pos_adj = jnp.pad(pos_table, ((0, 0), (0, POSW - EMBED))).reshape(-1)
    out = _emb(tok_pad, idx, pos_adj)
    return out[:, :, :EMBED]
